# baseline (speedup 1.0000x reference)
import jax
import jax.numpy as jnp
from jax import lax
from jax.experimental import pallas as pl
from jax.experimental.pallas import tpu as pltpu
from jax.experimental.pallas import tpu_sc as plsc

VOCAB = 100000
EMB = 32
B = 16384

NC = 2
NS = 16
CB = 512
QB = 2048
LG = 4
LT = 20
VAL = 99968
VCHUNK = 640
PART = NS * VCHUNK
NFULL = 9
TAIL = VAL - NFULL * PART

_NCOLS = [1] * 6 + [LG, LT]


def _body(x0, x1, x2, x3, x4, x5, xg, xt,
          w0, w1, w2, w3, w4, w5, wg, wt,
          t0, t1, t2, t3, t4, t5, tg, tt, outT,
          sh_tab, sh_out, row_v, tail_v, idx_v, out_v, sem):
    tables = [w0, w1, w2, w3, w4, w5, wg, wt]
    tails = [t0, t1, t2, t3, t4, t5, tg, tt]
    xone = [x0, x1, x2, x3, x4, x5]
    core = lax.axis_index("c")
    sub = lax.axis_index("s")
    crow = pl.multiple_of(core * NS, NS)
    cdyn = core * NS + sub

    for f in range(8):
        ncols = _NCOLS[f]
        def stage_part(p, _, f=f):
            pg = pl.multiple_of(p * PART + sub * VCHUNK, 128)
            so = pl.multiple_of(sub * VCHUNK, 128)
            pltpu.sync_copy(
                tables[f].at[pl.ds(crow, NS), pl.ds(pg, VCHUNK)],
                sh_tab.at[:, pl.ds(so, VCHUNK)])
            plsc.subcore_barrier()
            pltpu.sync_copy(
                sh_tab.at[sub, pl.ds(0, PART)],
                row_v.at[pl.ds(pl.multiple_of(p * PART, 128), PART)])
            plsc.subcore_barrier()
            return _

        lax.fori_loop(0, NFULL, stage_part, None)

        @pl.when(sub < 12)
        def _tail_full(f=f):
            pg = pl.multiple_of(NFULL * PART + sub * VCHUNK, 128)
            so = pl.multiple_of(sub * VCHUNK, 128)
            pltpu.sync_copy(
                tables[f].at[pl.ds(crow, NS), pl.ds(pg, VCHUNK)],
                sh_tab.at[:, pl.ds(so, VCHUNK)])

        @pl.when(sub == 12)
        def _tail_last(f=f):
            pltpu.sync_copy(
                tables[f].at[pl.ds(crow, NS), pl.ds(VAL - 128, 128)],
                sh_tab.at[:, pl.ds(12 * VCHUNK, 128)])

        plsc.subcore_barrier()
        pltpu.sync_copy(sh_tab.at[sub, pl.ds(0, TAIL)],
                        row_v.at[pl.ds(NFULL * PART, TAIL)])
        plsc.subcore_barrier()

        pltpu.sync_copy(tails[f], tail_v)
        rows16 = jnp.full((16,), cdyn, jnp.int32)
        for j in range(2):
            cols = lax.iota(jnp.int32, 16) + j * 16
            row_v[pl.ds(VAL + j * 16, 16)] = plsc.load_gather(
                tail_v, [rows16, cols])

        for q in range(B // QB):
            def bchunk(bc, _, f=f, ncols=ncols, q=q):
                b0 = pl.multiple_of(q * QB + bc * CB, CB)
                if ncols == 1:
                    pltpu.sync_copy(xone[f].at[pl.ds(b0, CB)], idx_v.at[0])
                elif ncols == LG:
                    pltpu.sync_copy(xg.at[:, pl.ds(b0, CB)],
                                    idx_v.at[pl.ds(0, LG)])
                else:
                    pltpu.sync_copy(xt.at[:, pl.ds(b0, CB)], idx_v)

                def group(g, _):
                    i0 = g * 16
                    if ncols == 1:
                        idxs = idx_v[0, pl.ds(i0, 16)]
                        out_v[pl.ds(i0, 16)] = plsc.load_gather(
                            row_v, [idxs])
                    else:
                        acc = jnp.zeros((16,), jnp.float32)
                        cnt = jnp.zeros((16,), jnp.float32)
                        for l in range(ncols):
                            idxs = idx_v[l, pl.ds(i0, 16)]
                            m = idxs > 0
                            gth = plsc.load_gather(row_v, [idxs])
                            acc = acc + jnp.where(m, gth, 0.0)
                            cnt = cnt + jnp.where(m, 1.0, 0.0)
                        out_v[pl.ds(i0, 16)] = acc / cnt
                    return _

                lax.fori_loop(0, CB // 16, group, None)
                pltpu.sync_copy(
                    out_v, sh_out.at[sub, pl.ds(
                        pl.multiple_of(bc * CB, CB), CB)])
                return _

            lax.fori_loop(0, QB // CB, bchunk, None)
            plsc.subcore_barrier()
            scol = pl.multiple_of(sub * (QB // NS), QB // NS)
            r0 = pl.multiple_of(EMB * f + core * NS, NS)
            pltpu.sync_copy(
                sh_out.at[:, pl.ds(scol, QB // NS)],
                outT.at[pl.ds(r0, NS),
                        pl.ds(pl.multiple_of(q * QB, QB) + scol, QB // NS)])
            plsc.subcore_barrier()


_sc_call = pl.kernel(
    _body,
    out_type=jax.ShapeDtypeStruct((8 * EMB, B), jnp.float32),
    mesh=plsc.VectorSubcoreMesh(core_axis_name="c", subcore_axis_name="s"),
    compiler_params=pltpu.CompilerParams(needs_layout_passes=False),
    scratch_types=[
        pltpu.VMEM_SHARED((NS, 10240), jnp.float32),
        pltpu.VMEM_SHARED((NS, QB), jnp.float32),
        pltpu.VMEM((VOCAB,), jnp.float32),
        pltpu.VMEM((EMB, EMB), jnp.float32),
        pltpu.VMEM((LT, CB), jnp.int32),
        pltpu.VMEM((CB,), jnp.float32),
        pltpu.SemaphoreType.DMA,
    ],
)


def kernel(x, w0, w1, w2, w3, w4, w5, w_genres, w_title):
    xT = x.T
    ws = [w0, w1, w2, w3, w4, w5, w_genres, w_title]
    wvs = [w.T for w in ws]
    tails = [wv[:, VAL:] for wv in wvs]
    outT = _sc_call(
        xT[0], xT[1], xT[2], xT[3], xT[4], xT[5], xT[6:10], xT[10:30],
        *wvs, *tails)
    return outT.T


# speedup vs baseline: 1.2511x; 1.2511x over previous
import jax
import jax.numpy as jnp
from jax import lax
from jax.experimental import pallas as pl
from jax.experimental.pallas import tpu as pltpu
from jax.experimental.pallas import tpu_sc as plsc

VOCAB = 100000
EMB = 32
B = 16384

NS = 16
BPT = B // NS
IC = 128
SC = 64
LG = 4
LT = 20


def _body(xT, w0, w1, w2, w3, w4, w5, wg, wt, out,
          idx_v, oh_v, g_v, t_v, pacc_v, stag_v,
          invg_v, c0g_v, invt_v, c0t_v, t0g_v, t0t_v, gsem):
    tabs = [w0, w1, w2, w3, w4, w5, wg, wt]
    core = lax.axis_index("c")
    sub = lax.axis_index("s")

    @pl.when(core == 1)
    def _pool_row0():
        pltpu.sync_copy(wg.at[0], t0g_v)
        pltpu.sync_copy(wt.at[0], t0t_v)

    def bchunk(ci, _):
        b0 = pl.multiple_of(sub * BPT + ci * IC, IC)
        pltpu.sync_copy(xT.at[:, pl.ds(b0, IC)], idx_v)
        for s in range(IC // SC):
            o = s * SC

            @pl.when(core == 0)
            def _sc0(o=o):
                cps = [pltpu.make_async_copy(
                    tabs[f].at[idx_v.at[f, pl.ds(o, SC)]], oh_v.at[f], gsem)
                    for f in range(4)]
                for cp in cps:
                    cp.start()
                for cp in cps:
                    cp.wait()
                def arow(i, _):
                    for f in range(4):
                        for dh in range(2):
                            d = dh * 16
                            stag_v[i, pl.ds(32 * f + d, 16)] = (
                                oh_v[f, i, pl.ds(d, 16)])
                    return _
                lax.fori_loop(0, SC, arow, None)

            @pl.when(core == 1)
            def _sc1(o=o):
                cps = [pltpu.make_async_copy(
                    tabs[4 + f].at[idx_v.at[4 + f, pl.ds(o, SC)]],
                    oh_v.at[f], gsem) for f in range(2)]
                cps += [pltpu.make_async_copy(
                    wg.at[idx_v.at[6 + l, pl.ds(o, SC)]], g_v.at[l], gsem)
                    for l in range(LG)]
                cps += [pltpu.make_async_copy(
                    wt.at[idx_v.at[10 + l, pl.ds(o, SC)]], t_v.at[l],
                    gsem) for l in range(LT // 2)]
                for cp in cps:
                    cp.start()

                for blk in range(SC // 16):
                    i0 = blk * 16
                    cg = jnp.zeros((16,), jnp.float32)
                    for l in range(LG):
                        v = idx_v[6 + l, pl.ds(o + i0, 16)]
                        cg = cg + jnp.where(v > 0, 1.0, 0.0)
                    invg_v[pl.ds(i0, 16)] = 1.0 / cg
                    c0g_v[pl.ds(i0, 16)] = float(LG) - cg
                    ct = jnp.zeros((16,), jnp.float32)
                    for l in range(LT):
                        v = idx_v[10 + l, pl.ds(o + i0, 16)]
                        ct = ct + jnp.where(v > 0, 1.0, 0.0)
                    invt_v[pl.ds(i0, 16)] = 1.0 / ct
                    c0t_v[pl.ds(i0, 16)] = float(LT) - ct

                for cp in cps:
                    cp.wait()

                def prow(i, _):
                    for dh in range(2):
                        d = dh * 16
                        a = t_v[0, i, pl.ds(d, 16)]
                        for l in range(1, LT // 2):
                            a = a + t_v[l, i, pl.ds(d, 16)]
                        pacc_v[i, pl.ds(d, 16)] = a
                    return _
                lax.fori_loop(0, SC, prow, None)

                cps2 = [pltpu.make_async_copy(
                    wt.at[idx_v.at[20 + l, pl.ds(o, SC)]], t_v.at[l],
                    gsem) for l in range(LT // 2)]
                for cp in cps2:
                    cp.start()
                for cp in cps2:
                    cp.wait()

                def frow(i, _):
                    invg = invg_v[pl.ds(i, 16)][0]
                    c0g = c0g_v[pl.ds(i, 16)][0]
                    invt = invt_v[pl.ds(i, 16)][0]
                    c0t = c0t_v[pl.ds(i, 16)][0]
                    for dh in range(2):
                        d = dh * 16
                        for f in range(2):
                            stag_v[i, pl.ds(32 * f + d, 16)] = (
                                oh_v[f, i, pl.ds(d, 16)])
                        ag = g_v[0, i, pl.ds(d, 16)]
                        for l in range(1, LG):
                            ag = ag + g_v[l, i, pl.ds(d, 16)]
                        stag_v[i, pl.ds(64 + d, 16)] = (
                            ag - c0g * t0g_v[pl.ds(d, 16)]) * invg
                        at = pacc_v[i, pl.ds(d, 16)]
                        for l in range(LT // 2):
                            at = at + t_v[l, i, pl.ds(d, 16)]
                        stag_v[i, pl.ds(96 + d, 16)] = (
                            at - c0t * t0t_v[pl.ds(d, 16)]) * invt
                    return _
                lax.fori_loop(0, SC, frow, None)

            ob = pl.multiple_of(b0 + o, 8)
            cl = pl.multiple_of(core * 128, 128)
            pltpu.sync_copy(stag_v, out.at[pl.ds(ob, SC), pl.ds(cl, 128)])
        return _

    lax.fori_loop(0, BPT // IC, bchunk, None)


_sc_call = pl.kernel(
    _body,
    out_type=jax.ShapeDtypeStruct((B, 8 * EMB), jnp.float32),
    mesh=plsc.VectorSubcoreMesh(core_axis_name="c", subcore_axis_name="s"),
    compiler_params=pltpu.CompilerParams(use_tc_tiling_on_sc=False),
    scratch_types=[
        pltpu.VMEM((32, IC), jnp.int32),
        pltpu.VMEM((4, SC, EMB), jnp.float32),
        pltpu.VMEM((LG, SC, EMB), jnp.float32),
        pltpu.VMEM((LT // 2, SC, EMB), jnp.float32),
        pltpu.VMEM((SC, EMB), jnp.float32),
        pltpu.VMEM((SC, 128), jnp.float32),
        pltpu.VMEM((SC + 16,), jnp.float32),
        pltpu.VMEM((SC + 16,), jnp.float32),
        pltpu.VMEM((SC + 16,), jnp.float32),
        pltpu.VMEM((SC + 16,), jnp.float32),
        pltpu.VMEM((EMB,), jnp.float32),
        pltpu.VMEM((EMB,), jnp.float32),
        pltpu.SemaphoreType.DMA,
    ],
)


def kernel(x, w0, w1, w2, w3, w4, w5, w_genres, w_title):
    xTp = jnp.pad(x, ((0, 0), (0, 2))).T
    return _sc_call(xTp, w0, w1, w2, w3, w4, w5, w_genres, w_title)
